# baseline (speedup 1.0000x reference)
import functools

import jax
import jax.numpy as jnp
from jax import lax
from jax.experimental import pallas as pl
from jax.experimental.pallas import tpu as pltpu
from jax.experimental.pallas import tpu_sc as plsc

N_NODES = 10000
N_EDGES = 160000
N_FACES = 160000
D0 = 128
D2P = 16

NC = 2
NS = 16
NW = NC * NS
CH = 128

ADJ_K = 80
ADJ_NNZ_PAD = NW * ADJ_K * CH
ADJ_NACC = 10240
ADJ_RPT = ADJ_NACC // NS
ADJ_TRASH = N_NODES

INC_K = 104
INC_NNZ_PAD = NS * INC_K * CH
INC_RANGE = N_FACES // NC
INC_NACC = 81920
INC_RPT = INC_NACC // NS
INC_TRASH = INC_RANGE


def _x1_pass_body(x_ref, w_ref, z_ref, s_ref):
    i = pl.program_id(0)
    x = x_ref[...]
    z_ref[...] = jnp.dot(x, w_ref[...], preferred_element_type=jnp.float32)
    @pl.when(i == 0)
    def _():
        s_ref[...] = jnp.zeros_like(s_ref)
    s_ref[...] += jnp.sum(x, axis=0, keepdims=True)


def _x1_pass(x1, w12p):
    rb = 8000
    grid = N_EDGES // rb
    return pl.pallas_call(
        _x1_pass_body,
        grid=(grid,),
        in_specs=[
            pl.BlockSpec((rb, D0), lambda i: (i, 0)),
            pl.BlockSpec((D0, D2P), lambda i: (0, 0)),
        ],
        out_specs=[
            pl.BlockSpec((rb, D2P), lambda i: (i, 0)),
            pl.BlockSpec((1, D0), lambda i: (0, 0)),
        ],
        out_shape=[
            jax.ShapeDtypeStruct((N_EDGES, D2P), jnp.float32),
            jax.ShapeDtypeStruct((1, D0), jnp.float32),
        ],
    )(x1, w12p)


def _mm_body(x_ref, w_ref, o_ref):
    o_ref[...] = jnp.dot(x_ref[...], w_ref[...],
                         preferred_element_type=jnp.float32)


def _mm(x, w):
    rb = 2000
    grid = x.shape[0] // rb
    return pl.pallas_call(
        _mm_body,
        grid=(grid,),
        in_specs=[
            pl.BlockSpec((rb, D0), lambda i: (i, 0)),
            pl.BlockSpec((D0, D0), lambda i: (0, 0)),
        ],
        out_specs=pl.BlockSpec((rb, D0), lambda i: (i, 0)),
        out_shape=jax.ShapeDtypeStruct((x.shape[0], D0), jnp.float32),
    )(x, w)


def _mid_body(a_ref, w_ref, o_ref):
    t = jax.nn.relu(a_ref[0] + a_ref[1])
    o_ref[...] = jnp.dot(t, w_ref[...], preferred_element_type=jnp.float32)


def _mid(a, w):
    rb = 2000
    grid = N_NODES // rb
    return pl.pallas_call(
        _mid_body,
        grid=(grid,),
        in_specs=[
            pl.BlockSpec((2, rb, D0), lambda i: (0, i, 0)),
            pl.BlockSpec((D0, D0), lambda i: (0, 0)),
        ],
        out_specs=pl.BlockSpec((rb, D0), lambda i: (i, 0)),
        out_shape=jax.ShapeDtypeStruct((N_NODES, D0), jnp.float32),
    )(a, w)


def _make_sc_spmm(table_rows, d, k_chunks, stage_k, slots, nacc, rpt, trash,
                  split_range, ch=CH):
    mesh = plsc.VectorSubcoreMesh(core_axis_name="c", subcore_axis_name="s")

    def body(table, src2d, dst2d, zeros, out, sidx_v, didx_v, rows_v, accum,
             *sems):
        sem_g = sems[:slots]
        sem_s = sems[slots:]
        c = lax.axis_index("c")
        s = lax.axis_index("s")

        def fire_g(u, chunk):
            pltpu.async_copy(table.at[sidx_v.at[chunk]], rows_v.at[u],
                             sem_g[u])

        def wait_g(u):
            pltpu.make_async_copy(table.at[sidx_v.at[0]], rows_v.at[u],
                                  sem_g[u]).wait()

        def fire_s(u, chunk):
            pltpu.async_copy(rows_v.at[u], accum.at[didx_v.at[chunk]],
                             sem_s[u], add=True)

        def wait_s(u):
            pltpu.make_async_copy(rows_v.at[u], accum.at[didx_v.at[0]],
                                  sem_s[u]).wait()

        pltpu.sync_copy(zeros, accum.at[pl.ds(s * rpt, rpt)])
        plsc.subcore_barrier()

        if split_range:
            row_base = s * k_chunks
        else:
            row_base = (s * NC + c) * k_chunks

        n_steps = stage_k // slots
        for stage in range(k_chunks // stage_k):
            row0 = row_base + stage * stage_k
            pltpu.sync_copy(src2d.at[pl.ds(row0, stage_k)], sidx_v)
            pltpu.sync_copy(dst2d.at[pl.ds(row0, stage_k)], didx_v)

            if split_range:
                lo = c * INC_RANGE
                def remap(j, _):
                    def remap16(q, _):
                        dv = didx_v[j, pl.ds(q * 16, 16)]
                        inr = (dv >= lo) & (dv < lo + INC_RANGE)
                        didx_v[j, pl.ds(q * 16, 16)] = jnp.where(
                            inr, dv - lo, trash + (dv & 1023))
                        return 0
                    return lax.fori_loop(0, ch // 16, remap16, 0)
                lax.fori_loop(0, stage_k, remap, 0)

            for u in range(slots):
                fire_g(u, u)

            def step(t, _):
                base = t * slots
                for u in range(slots):
                    wait_g(u)
                    fire_s(u, base + u)
                for u in range(slots):
                    wait_s(u)
                    @pl.when(t < n_steps - 1)
                    def _():
                        fire_g(u, base + slots + u)
                return 0
            lax.fori_loop(0, n_steps, step, 0)

        plsc.subcore_barrier()

        pltpu.sync_copy(accum.at[pl.ds(s * rpt, rpt)],
                        out.at[pl.ds(c * nacc + s * rpt, rpt)])

    return functools.partial(
        pl.kernel,
        out_type=jax.ShapeDtypeStruct((NC * nacc, d), jnp.float32),
        mesh=mesh,
        compiler_params=pltpu.CompilerParams(
            use_tc_tiling_on_sc=(d == D0)),
        scratch_types=[
            pltpu.VMEM((stage_k, ch), jnp.int32),
            pltpu.VMEM((stage_k, ch), jnp.int32),
            pltpu.VMEM((slots, ch, d), jnp.float32),
            pltpu.VMEM_SHARED((nacc, d), jnp.float32),
        ] + [pltpu.SemaphoreType.DMA] * (2 * slots),
    )(body)


_adj_spmm = _make_sc_spmm(N_NODES, D0, ADJ_K * 2, 40, 4, ADJ_NACC, ADJ_RPT,
                          ADJ_TRASH, split_range=False, ch=64)
_inc_spmm = _make_sc_spmm(N_EDGES, D2P, INC_K, INC_K, 8, INC_NACC, INC_RPT,
                          INC_TRASH, split_range=True)


def _final_body(a_ref, b_ref, cs1_ref, l0_ref, b0_ref, l1_ref, b1_ref,
                l2_ref, b2_ref, o_ref, s0_ref, s2_ref):
    i = pl.program_id(0)
    n = pl.num_programs(0)

    @pl.when(i == 0)
    def _():
        s0_ref[...] = jnp.zeros_like(s0_ref)
        s2_ref[...] = jnp.zeros_like(s2_ref)

    x0 = jax.nn.relu(a_ref[0] + a_ref[1])
    s0_ref[...] += jnp.sum(x0, axis=0, keepdims=True)
    x2 = jax.nn.relu(b_ref[...])
    s2_ref[...] += jnp.sum(x2, axis=(0, 1)).reshape(1, D2P)

    @pl.when(i == n - 1)
    def _():
        y0 = jnp.dot(s0_ref[...] * (1.0 / N_NODES), l0_ref[...],
                     preferred_element_type=jnp.float32) + b0_ref[...]
        y1 = jnp.dot(cs1_ref[...] * (1.0 / N_EDGES), l1_ref[...],
                     preferred_element_type=jnp.float32) + b1_ref[...]
        y2 = jnp.dot(s2_ref[...] * (1.0 / N_FACES), l2_ref[...],
                     preferred_element_type=jnp.float32) + b2_ref[...]
        o_ref[...] = y0 + y1 + y2


def _final(a, b, cs1, l0p, b0p, l1p, b1p, l2p, b2p):
    grid = 10
    rb0 = N_NODES // grid
    rb2 = INC_RANGE // grid
    return pl.pallas_call(
        _final_body,
        grid=(grid,),
        in_specs=[
            pl.BlockSpec((2, rb0, D0), lambda i: (0, i, 0)),
            pl.BlockSpec((2, rb2, D2P), lambda i: (0, i, 0)),
            pl.BlockSpec((1, D0), lambda i: (0, 0)),
            pl.BlockSpec((D0, 128), lambda i: (0, 0)),
            pl.BlockSpec((1, 128), lambda i: (0, 0)),
            pl.BlockSpec((D0, 128), lambda i: (0, 0)),
            pl.BlockSpec((1, 128), lambda i: (0, 0)),
            pl.BlockSpec((D2P, 128), lambda i: (0, 0)),
            pl.BlockSpec((1, 128), lambda i: (0, 0)),
        ],
        out_specs=pl.BlockSpec((1, 128), lambda i: (0, 0)),
        out_shape=jax.ShapeDtypeStruct((1, 128), jnp.float32),
        scratch_shapes=[
            pltpu.VMEM((1, D0), jnp.float32),
            pltpu.VMEM((1, D2P), jnp.float32),
        ],
    )(a, b, cs1, l0p, b0p, l1p, b1p, l2p, b2p)


def _pad_idx(src, dst, nnz_pad, trash_dst, trash_spread, ch=CH):
    n = src.shape[0]
    pad = nnz_pad - n
    src_p = jnp.concatenate([src.astype(jnp.int32),
                             jnp.zeros((pad,), jnp.int32)])
    trash = trash_dst + jnp.arange(pad, dtype=jnp.int32) % trash_spread
    dst_p = jnp.concatenate([dst.astype(jnp.int32), trash])
    return src_p.reshape(-1, ch), dst_p.reshape(-1, ch)


def kernel(x_0, x_1, adjacency_0, incidence_2_t,
           W0_0, W12_0, W0_1, W12_1,
           lin0_w, lin0_b, lin1_w, lin1_b, lin2_w, lin2_b):
    f32 = jnp.float32

    w12p = jnp.zeros((D0, D2P), f32).at[:, :5].set(W12_1)
    z1p, cs1 = _x1_pass(x_1, w12p)

    inc_src, inc_dst = _pad_idx(incidence_2_t[1], incidence_2_t[0],
                                INC_NNZ_PAD, N_FACES, 1)
    inc_zeros = jnp.zeros((INC_RPT, D2P), f32)
    x2acc = _inc_spmm(z1p, inc_src, inc_dst, inc_zeros)

    adj_src, adj_dst = _pad_idx(adjacency_0[1], adjacency_0[0],
                                ADJ_NNZ_PAD, ADJ_TRASH, 192, ch=64)
    adj_zeros = jnp.zeros((ADJ_RPT, D0), f32)

    z0 = _mm(x_0, W0_0)
    a1 = _adj_spmm(z0, adj_src, adj_dst, adj_zeros)
    h = _mid(a1.reshape(2, ADJ_NACC, D0), W0_1)
    a2 = _adj_spmm(h, adj_src, adj_dst, adj_zeros)

    def padw(w, rows):
        wp = jnp.zeros((rows, 128), f32)
        return wp.at[:w.shape[0], :2].set(w)

    def padb(b):
        return jnp.zeros((1, 128), f32).at[0, :2].set(b)

    out = _final(a2.reshape(2, ADJ_NACC, D0),
                 x2acc.reshape(2, INC_NACC, D2P),
                 cs1,
                 padw(lin0_w, D0), padb(lin0_b),
                 padw(lin1_w, D0), padb(lin1_b),
                 padw(lin2_w, D2P), padb(lin2_b))
    return out[0, :2]


# speedup vs baseline: 1.2246x; 1.2246x over previous
import functools

import jax
import jax.numpy as jnp
from jax import lax
from jax.experimental import pallas as pl
from jax.experimental.pallas import tpu as pltpu
from jax.experimental.pallas import tpu_sc as plsc

N_NODES = 10000
N_EDGES = 160000
N_FACES = 160000
D0 = 128
D2P = 16

NC = 2
NS = 16
NW = NC * NS
CH = 128

ADJ_K = 80
ADJ_NNZ_PAD = NW * ADJ_K * CH
ADJ_NACC = 10240
ADJ_RPT = ADJ_NACC // NS
ADJ_TRASH = N_NODES

INC_K = 104
INC_NNZ_PAD = NS * INC_K * CH
INC_RANGE = N_FACES // NC
INC_NACC = 81920
INC_RPT = INC_NACC // NS
INC_TRASH = INC_RANGE


def _x1_pass_body(x_ref, w_ref, z_ref, s_ref):
    i = pl.program_id(0)
    x = x_ref[...]
    z_ref[...] = jnp.dot(x, w_ref[...], preferred_element_type=jnp.float32)
    @pl.when(i == 0)
    def _():
        s_ref[...] = jnp.zeros_like(s_ref)
    s_ref[...] += jnp.sum(x, axis=0, keepdims=True)


def _x1_pass(x1, w12p):
    rb = 8000
    grid = N_EDGES // rb
    return pl.pallas_call(
        _x1_pass_body,
        grid=(grid,),
        in_specs=[
            pl.BlockSpec((rb, D0), lambda i: (i, 0)),
            pl.BlockSpec((D0, D2P), lambda i: (0, 0)),
        ],
        out_specs=[
            pl.BlockSpec((rb, D2P), lambda i: (i, 0)),
            pl.BlockSpec((1, D0), lambda i: (0, 0)),
        ],
        out_shape=[
            jax.ShapeDtypeStruct((N_EDGES, D2P), jnp.float32),
            jax.ShapeDtypeStruct((1, D0), jnp.float32),
        ],
    )(x1, w12p)


def _mm_body(x_ref, w_ref, o_ref):
    o_ref[...] = jnp.dot(x_ref[...], w_ref[...],
                         preferred_element_type=jnp.float32)


def _mm(x, w):
    rb = 2000
    grid = x.shape[0] // rb
    return pl.pallas_call(
        _mm_body,
        grid=(grid,),
        in_specs=[
            pl.BlockSpec((rb, D0), lambda i: (i, 0)),
            pl.BlockSpec((D0, D0), lambda i: (0, 0)),
        ],
        out_specs=pl.BlockSpec((rb, D0), lambda i: (i, 0)),
        out_shape=jax.ShapeDtypeStruct((x.shape[0], D0), jnp.float32),
    )(x, w)


def _mid_body(a_ref, w_ref, o_ref):
    t = jax.nn.relu(a_ref[0] + a_ref[1])
    o_ref[...] = jnp.dot(t, w_ref[...], preferred_element_type=jnp.float32)


def _mid(a, w):
    rb = 2000
    grid = N_NODES // rb
    return pl.pallas_call(
        _mid_body,
        grid=(grid,),
        in_specs=[
            pl.BlockSpec((2, rb, D0), lambda i: (0, i, 0)),
            pl.BlockSpec((D0, D0), lambda i: (0, 0)),
        ],
        out_specs=pl.BlockSpec((rb, D0), lambda i: (i, 0)),
        out_shape=jax.ShapeDtypeStruct((N_NODES, D0), jnp.float32),
    )(a, w)


def _make_sc_spmm(table_rows, d, k_chunks, stage_k, slots, nacc, rpt, trash,
                  split_range, ch=CH):
    mesh = plsc.VectorSubcoreMesh(core_axis_name="c", subcore_axis_name="s")

    def body(table, src2d, dst2d, zeros, out, sidx_v, didx_v, rows_v, accum,
             *sems):
        sem_g = sems[:slots]
        sem_s = sems[slots:]
        c = lax.axis_index("c")
        s = lax.axis_index("s")

        def fire_g(u, chunk):
            pltpu.async_copy(table.at[sidx_v.at[chunk]], rows_v.at[u],
                             sem_g[u])

        def wait_g(u):
            pltpu.make_async_copy(table.at[sidx_v.at[0]], rows_v.at[u],
                                  sem_g[u]).wait()

        def fire_s(u, chunk):
            pltpu.async_copy(rows_v.at[u], accum.at[didx_v.at[chunk]],
                             sem_s[u], add=True)

        def wait_s(u):
            pltpu.make_async_copy(rows_v.at[u], accum.at[didx_v.at[0]],
                                  sem_s[u]).wait()

        pltpu.sync_copy(zeros, accum.at[pl.ds(s * rpt, rpt)])
        plsc.subcore_barrier()

        if split_range:
            row_base = s * k_chunks
        else:
            row_base = (s * NC + c) * k_chunks

        n_steps = stage_k // slots
        for stage in range(k_chunks // stage_k):
            row0 = row_base + stage * stage_k
            pltpu.sync_copy(src2d.at[pl.ds(row0, stage_k)], sidx_v)
            pltpu.sync_copy(dst2d.at[pl.ds(row0, stage_k)], didx_v)

            if split_range:
                lo = c * INC_RANGE
                def remap(j, _):
                    def remap16(q, _):
                        dv = didx_v[j, pl.ds(q * 16, 16)]
                        inr = (dv >= lo) & (dv < lo + INC_RANGE)
                        didx_v[j, pl.ds(q * 16, 16)] = jnp.where(
                            inr, dv - lo, trash + (dv & 1023))
                        return 0
                    return lax.fori_loop(0, ch // 16, remap16, 0)
                lax.fori_loop(0, stage_k, remap, 0)

            for u in range(slots):
                fire_g(u, u)

            def step(t, _):
                base = t * slots
                for u in range(slots):
                    wait_g(u)
                    fire_s(u, base + u)
                for u in range(slots):
                    wait_s(u)
                    @pl.when(t < n_steps - 1)
                    def _():
                        fire_g(u, base + slots + u)
                return 0
            lax.fori_loop(0, n_steps, step, 0)

        plsc.subcore_barrier()

        pltpu.sync_copy(accum.at[pl.ds(s * rpt, rpt)],
                        out.at[pl.ds(c * nacc + s * rpt, rpt)])

    return functools.partial(
        pl.kernel,
        out_type=jax.ShapeDtypeStruct((NC * nacc, d), jnp.float32),
        mesh=mesh,
        compiler_params=pltpu.CompilerParams(
            use_tc_tiling_on_sc=(d == D0)),
        scratch_types=[
            pltpu.VMEM((stage_k, ch), jnp.int32),
            pltpu.VMEM((stage_k, ch), jnp.int32),
            pltpu.VMEM((slots, ch, d), jnp.float32),
            pltpu.VMEM_SHARED((nacc, d), jnp.float32),
        ] + [pltpu.SemaphoreType.DMA] * (2 * slots),
    )(body)


_adj_spmm = _make_sc_spmm(N_NODES, D0, ADJ_K, 40, 2, ADJ_NACC, ADJ_RPT,
                          ADJ_TRASH, split_range=False)
_inc_spmm = _make_sc_spmm(N_EDGES, D2P, INC_K, INC_K, 8, INC_NACC, INC_RPT,
                          INC_TRASH, split_range=True)


def _final_body(a_ref, b_ref, cs1_ref, l0_ref, b0_ref, l1_ref, b1_ref,
                l2_ref, b2_ref, o_ref, s0_ref, s2_ref):
    i = pl.program_id(0)
    n = pl.num_programs(0)

    @pl.when(i == 0)
    def _():
        s0_ref[...] = jnp.zeros_like(s0_ref)
        s2_ref[...] = jnp.zeros_like(s2_ref)

    x0 = jax.nn.relu(a_ref[0] + a_ref[1])
    s0_ref[...] += jnp.sum(x0, axis=0, keepdims=True)
    x2 = jax.nn.relu(b_ref[...])
    s2_ref[...] += jnp.sum(x2, axis=(0, 1)).reshape(1, D2P)

    @pl.when(i == n - 1)
    def _():
        y0 = jnp.dot(s0_ref[...] * (1.0 / N_NODES), l0_ref[...],
                     preferred_element_type=jnp.float32) + b0_ref[...]
        y1 = jnp.dot(cs1_ref[...] * (1.0 / N_EDGES), l1_ref[...],
                     preferred_element_type=jnp.float32) + b1_ref[...]
        y2 = jnp.dot(s2_ref[...] * (1.0 / N_FACES), l2_ref[...],
                     preferred_element_type=jnp.float32) + b2_ref[...]
        o_ref[...] = y0 + y1 + y2


def _final(a, b, cs1, l0p, b0p, l1p, b1p, l2p, b2p):
    grid = 10
    rb0 = N_NODES // grid
    rb2 = INC_RANGE // grid
    return pl.pallas_call(
        _final_body,
        grid=(grid,),
        in_specs=[
            pl.BlockSpec((2, rb0, D0), lambda i: (0, i, 0)),
            pl.BlockSpec((2, rb2, D2P), lambda i: (0, i, 0)),
            pl.BlockSpec((1, D0), lambda i: (0, 0)),
            pl.BlockSpec((D0, 128), lambda i: (0, 0)),
            pl.BlockSpec((1, 128), lambda i: (0, 0)),
            pl.BlockSpec((D0, 128), lambda i: (0, 0)),
            pl.BlockSpec((1, 128), lambda i: (0, 0)),
            pl.BlockSpec((D2P, 128), lambda i: (0, 0)),
            pl.BlockSpec((1, 128), lambda i: (0, 0)),
        ],
        out_specs=pl.BlockSpec((1, 128), lambda i: (0, 0)),
        out_shape=jax.ShapeDtypeStruct((1, 128), jnp.float32),
        scratch_shapes=[
            pltpu.VMEM((1, D0), jnp.float32),
            pltpu.VMEM((1, D2P), jnp.float32),
        ],
    )(a, b, cs1, l0p, b0p, l1p, b1p, l2p, b2p)


def _pad_idx(src, dst, nnz_pad, trash_dst, trash_spread, ch=CH):
    n = src.shape[0]
    pad = nnz_pad - n
    src_p = jnp.concatenate([src.astype(jnp.int32),
                             jnp.zeros((pad,), jnp.int32)])
    trash = trash_dst + jnp.arange(pad, dtype=jnp.int32) % trash_spread
    dst_p = jnp.concatenate([dst.astype(jnp.int32), trash])
    return src_p.reshape(-1, ch), dst_p.reshape(-1, ch)


def kernel(x_0, x_1, adjacency_0, incidence_2_t,
           W0_0, W12_0, W0_1, W12_1,
           lin0_w, lin0_b, lin1_w, lin1_b, lin2_w, lin2_b):
    f32 = jnp.float32

    w12p = jnp.zeros((D0, D2P), f32).at[:, :5].set(W12_1)
    z1p, cs1 = _x1_pass(x_1, w12p)

    inc_src, inc_dst = _pad_idx(incidence_2_t[1], incidence_2_t[0],
                                INC_NNZ_PAD, N_FACES, 1)
    inc_zeros = jnp.zeros((INC_RPT, D2P), f32)
    x2acc = _inc_spmm(z1p, inc_src, inc_dst, inc_zeros)

    adj_src, adj_dst = _pad_idx(adjacency_0[1], adjacency_0[0],
                                ADJ_NNZ_PAD, ADJ_TRASH, 192)
    adj_zeros = jnp.zeros((ADJ_RPT, D0), f32)

    z0 = _mm(x_0, W0_0)
    a1 = _adj_spmm(z0, adj_src, adj_dst, adj_zeros)
    h = _mid(a1.reshape(2, ADJ_NACC, D0), W0_1)
    a2 = _adj_spmm(h, adj_src, adj_dst, adj_zeros)

    def padw(w, rows):
        wp = jnp.zeros((rows, 128), f32)
        return wp.at[:w.shape[0], :2].set(w)

    def padb(b):
        return jnp.zeros((1, 128), f32).at[0, :2].set(b)

    out = _final(a2.reshape(2, ADJ_NACC, D0),
                 x2acc.reshape(2, INC_NACC, D2P),
                 cs1,
                 padw(lin0_w, D0), padb(lin0_b),
                 padw(lin1_w, D0), padb(lin1_b),
                 padw(lin2_w, D2P), padb(lin2_b))
    return out[0, :2]


# speedup vs baseline: 1.6600x; 1.3555x over previous
import functools

import jax
import jax.numpy as jnp
from jax import lax
from jax.experimental import pallas as pl
from jax.experimental.pallas import tpu as pltpu
from jax.experimental.pallas import tpu_sc as plsc

N_NODES = 10000
N_EDGES = 160000
N_FACES = 160000
D0 = 128
D2P = 16

NC = 2
NS = 16
NW = NC * NS
CH = 128

ADJ_K = 80
ADJ_NNZ_PAD = NW * ADJ_K * CH
ADJ_NACC = 10240
ADJ_RPT = ADJ_NACC // NS
ADJ_TRASH = N_NODES

INC_K = 104
INC_NNZ_PAD = NS * INC_K * CH
INC_RANGE = N_FACES // NC
INC_NACC = 81920
INC_RPT = INC_NACC // NS
INC_TRASH = INC_RANGE


def _x1_pass_body(x_ref, w_ref, z_ref, s_ref):
    i = pl.program_id(0)
    x = x_ref[...]
    z_ref[...] = jnp.dot(x, w_ref[...], preferred_element_type=jnp.float32)
    @pl.when(i == 0)
    def _():
        s_ref[...] = jnp.zeros_like(s_ref)
    s_ref[...] += jnp.sum(x, axis=0, keepdims=True)


def _x1_pass(x1, w12p):
    rb = 8000
    grid = N_EDGES // rb
    return pl.pallas_call(
        _x1_pass_body,
        grid=(grid,),
        in_specs=[
            pl.BlockSpec((rb, D0), lambda i: (i, 0)),
            pl.BlockSpec((D0, D2P), lambda i: (0, 0)),
        ],
        out_specs=[
            pl.BlockSpec((rb, D2P), lambda i: (i, 0)),
            pl.BlockSpec((1, D0), lambda i: (0, 0)),
        ],
        out_shape=[
            jax.ShapeDtypeStruct((N_EDGES, D2P), jnp.float32),
            jax.ShapeDtypeStruct((1, D0), jnp.float32),
        ],
    )(x1, w12p)


def _mm_body(x_ref, w_ref, o_ref):
    o_ref[...] = jnp.dot(x_ref[...], w_ref[...],
                         preferred_element_type=jnp.float32
                         ).astype(jnp.bfloat16)


def _mm(x, w):
    rb = 2000
    grid = x.shape[0] // rb
    return pl.pallas_call(
        _mm_body,
        grid=(grid,),
        in_specs=[
            pl.BlockSpec((rb, D0), lambda i: (i, 0)),
            pl.BlockSpec((D0, D0), lambda i: (0, 0)),
        ],
        out_specs=pl.BlockSpec((rb, D0), lambda i: (i, 0)),
        out_shape=jax.ShapeDtypeStruct((x.shape[0], D0), jnp.bfloat16),
    )(x, w)


def _mid_body(a_ref, w_ref, o_ref):
    t = jax.nn.relu(a_ref[0].astype(jnp.float32) + a_ref[1].astype(jnp.float32))
    o_ref[...] = jnp.dot(t, w_ref[...], preferred_element_type=jnp.float32
                         ).astype(jnp.bfloat16)


def _mid(a, w):
    rb = 2000
    grid = N_NODES // rb
    return pl.pallas_call(
        _mid_body,
        grid=(grid,),
        in_specs=[
            pl.BlockSpec((2, rb, D0), lambda i: (0, i, 0)),
            pl.BlockSpec((D0, D0), lambda i: (0, 0)),
        ],
        out_specs=pl.BlockSpec((rb, D0), lambda i: (i, 0)),
        out_shape=jax.ShapeDtypeStruct((N_NODES, D0), jnp.bfloat16),
    )(a, w)


def _make_sc_spmm(table_rows, d, k_chunks, stage_k, slots, nacc, rpt, trash,
                  split_range, ch=CH, dtype=jnp.float32, tc_tiling=None):
    mesh = plsc.VectorSubcoreMesh(core_axis_name="c", subcore_axis_name="s")

    def body(table, src2d, dst2d, zeros, out, sidx_v, didx_v, rows_v, accum,
             *sems):
        sem_g = sems[:slots]
        sem_s = sems[slots:]
        c = lax.axis_index("c")
        s = lax.axis_index("s")

        def fire_g(u, chunk):
            pltpu.async_copy(table.at[sidx_v.at[chunk]], rows_v.at[u],
                             sem_g[u])

        def wait_g(u):
            pltpu.make_async_copy(table.at[sidx_v.at[0]], rows_v.at[u],
                                  sem_g[u]).wait()

        def fire_s(u, chunk):
            pltpu.async_copy(rows_v.at[u], accum.at[didx_v.at[chunk]],
                             sem_s[u], add=True)

        def wait_s(u):
            pltpu.make_async_copy(rows_v.at[u], accum.at[didx_v.at[0]],
                                  sem_s[u]).wait()

        pltpu.sync_copy(zeros, accum.at[pl.ds(s * rpt, rpt)])
        plsc.subcore_barrier()

        if split_range:
            row_base = s * k_chunks
        else:
            row_base = (s * NC + c) * k_chunks

        n_steps = stage_k // slots
        for stage in range(k_chunks // stage_k):
            row0 = row_base + stage * stage_k
            pltpu.sync_copy(src2d.at[pl.ds(row0, stage_k)], sidx_v)
            pltpu.sync_copy(dst2d.at[pl.ds(row0, stage_k)], didx_v)

            if split_range:
                lo = c * INC_RANGE
                def remap(j, _):
                    def remap16(q, _):
                        dv = didx_v[j, pl.ds(q * 16, 16)]
                        inr = (dv >= lo) & (dv < lo + INC_RANGE)
                        didx_v[j, pl.ds(q * 16, 16)] = jnp.where(
                            inr, dv - lo, trash + (dv & 1023))
                        return 0
                    return lax.fori_loop(0, ch // 16, remap16, 0)
                lax.fori_loop(0, stage_k, remap, 0)

            for u in range(slots):
                fire_g(u, u)

            def step(t, _):
                base = t * slots
                for u in range(slots):
                    wait_g(u)
                    fire_s(u, base + u)
                for u in range(slots):
                    wait_s(u)
                    @pl.when(t < n_steps - 1)
                    def _():
                        fire_g(u, base + slots + u)
                return 0
            lax.fori_loop(0, n_steps, step, 0)

        plsc.subcore_barrier()

        pltpu.sync_copy(accum.at[pl.ds(s * rpt, rpt)],
                        out.at[pl.ds(c * nacc + s * rpt, rpt)])

    return functools.partial(
        pl.kernel,
        out_type=jax.ShapeDtypeStruct((NC * nacc, d), dtype),
        mesh=mesh,
        compiler_params=pltpu.CompilerParams(
            use_tc_tiling_on_sc=(d == D0 and dtype == jnp.float32
                                 if tc_tiling is None else tc_tiling)),
        scratch_types=[
            pltpu.VMEM((stage_k, ch), jnp.int32),
            pltpu.VMEM((stage_k, ch), jnp.int32),
            pltpu.VMEM((slots, ch, d), dtype),
            pltpu.VMEM_SHARED((nacc, d), dtype),
        ] + [pltpu.SemaphoreType.DMA] * (2 * slots),
    )(body)


_adj_spmm = _make_sc_spmm(N_NODES, D0, ADJ_K, 80, 4, ADJ_NACC, ADJ_RPT,
                          ADJ_TRASH, split_range=False, dtype=jnp.bfloat16)
_inc_spmm = _make_sc_spmm(N_EDGES, D2P, INC_K, INC_K, 8, INC_NACC, INC_RPT,
                          INC_TRASH, split_range=True)


def _final_body(a_ref, b_ref, cs1_ref, l0_ref, b0_ref, l1_ref, b1_ref,
                l2_ref, b2_ref, o_ref, s0_ref, s2_ref):
    i = pl.program_id(0)
    n = pl.num_programs(0)

    @pl.when(i == 0)
    def _():
        s0_ref[...] = jnp.zeros_like(s0_ref)
        s2_ref[...] = jnp.zeros_like(s2_ref)

    x0 = jax.nn.relu(a_ref[0].astype(jnp.float32)
                     + a_ref[1].astype(jnp.float32))
    s0_ref[...] += jnp.sum(x0, axis=0, keepdims=True)
    x2 = jax.nn.relu(b_ref[...])
    s2_ref[...] += jnp.sum(x2, axis=(0, 1)).reshape(1, D2P)

    @pl.when(i == n - 1)
    def _():
        y0 = jnp.dot(s0_ref[...] * (1.0 / N_NODES), l0_ref[...],
                     preferred_element_type=jnp.float32) + b0_ref[...]
        y1 = jnp.dot(cs1_ref[...] * (1.0 / N_EDGES), l1_ref[...],
                     preferred_element_type=jnp.float32) + b1_ref[...]
        y2 = jnp.dot(s2_ref[...] * (1.0 / N_FACES), l2_ref[...],
                     preferred_element_type=jnp.float32) + b2_ref[...]
        o_ref[...] = y0 + y1 + y2


def _final(a, b, cs1, l0p, b0p, l1p, b1p, l2p, b2p):
    grid = 10
    rb0 = N_NODES // grid
    rb2 = INC_RANGE // grid
    return pl.pallas_call(
        _final_body,
        grid=(grid,),
        in_specs=[
            pl.BlockSpec((2, rb0, D0), lambda i: (0, i, 0)),
            pl.BlockSpec((2, rb2, D2P), lambda i: (0, i, 0)),
            pl.BlockSpec((1, D0), lambda i: (0, 0)),
            pl.BlockSpec((D0, 128), lambda i: (0, 0)),
            pl.BlockSpec((1, 128), lambda i: (0, 0)),
            pl.BlockSpec((D0, 128), lambda i: (0, 0)),
            pl.BlockSpec((1, 128), lambda i: (0, 0)),
            pl.BlockSpec((D2P, 128), lambda i: (0, 0)),
            pl.BlockSpec((1, 128), lambda i: (0, 0)),
        ],
        out_specs=pl.BlockSpec((1, 128), lambda i: (0, 0)),
        out_shape=jax.ShapeDtypeStruct((1, 128), jnp.float32),
        scratch_shapes=[
            pltpu.VMEM((1, D0), jnp.float32),
            pltpu.VMEM((1, D2P), jnp.float32),
        ],
    )(a, b, cs1, l0p, b0p, l1p, b1p, l2p, b2p)


def _pad_idx(src, dst, nnz_pad, trash_dst, trash_spread, ch=CH):
    n = src.shape[0]
    pad = nnz_pad - n
    src_p = jnp.concatenate([src.astype(jnp.int32),
                             jnp.zeros((pad,), jnp.int32)])
    trash = trash_dst + jnp.arange(pad, dtype=jnp.int32) % trash_spread
    dst_p = jnp.concatenate([dst.astype(jnp.int32), trash])
    return src_p.reshape(-1, ch), dst_p.reshape(-1, ch)


def kernel(x_0, x_1, adjacency_0, incidence_2_t,
           W0_0, W12_0, W0_1, W12_1,
           lin0_w, lin0_b, lin1_w, lin1_b, lin2_w, lin2_b):
    f32 = jnp.float32

    w12p = jnp.zeros((D0, D2P), f32).at[:, :5].set(W12_1)
    z1p, cs1 = _x1_pass(x_1, w12p)

    inc_src, inc_dst = _pad_idx(incidence_2_t[1], incidence_2_t[0],
                                INC_NNZ_PAD, N_FACES, 1)
    inc_zeros = jnp.zeros((INC_RPT, D2P), f32)
    x2acc = _inc_spmm(z1p, inc_src, inc_dst, inc_zeros)

    adj_src, adj_dst = _pad_idx(adjacency_0[1], adjacency_0[0],
                                ADJ_NNZ_PAD, ADJ_TRASH, 192)
    adj_zeros = jnp.zeros((ADJ_RPT, D0), jnp.bfloat16)

    z0 = _mm(x_0, W0_0)
    a1 = _adj_spmm(z0, adj_src, adj_dst, adj_zeros)
    h = _mid(a1.reshape(2, ADJ_NACC, D0), W0_1)
    a2 = _adj_spmm(h, adj_src, adj_dst, adj_zeros)

    def padw(w, rows):
        wp = jnp.zeros((rows, 128), f32)
        return wp.at[:w.shape[0], :2].set(w)

    def padb(b):
        return jnp.zeros((1, 128), f32).at[0, :2].set(b)

    out = _final(a2.reshape(2, ADJ_NACC, D0),
                 x2acc.reshape(2, INC_NACC, D2P),
                 cs1,
                 padw(lin0_w, D0), padb(lin0_b),
                 padw(lin1_w, D0), padb(lin1_b),
                 padw(lin2_w, D2P), padb(lin2_b))
    return out[0, :2]


# speedup vs baseline: 1.6866x; 1.0161x over previous
import functools

import jax
import jax.numpy as jnp
from jax import lax
from jax.experimental import pallas as pl
from jax.experimental.pallas import tpu as pltpu
from jax.experimental.pallas import tpu_sc as plsc

N_NODES = 10000
N_EDGES = 160000
N_FACES = 160000
D0 = 128
D2P = 16

NC = 2
NS = 16
NW = NC * NS
CH = 128

ADJ_K = 80
ADJ_NNZ_PAD = NW * ADJ_K * CH
ADJ_NACC = 10240
ADJ_RPT = ADJ_NACC // NS
ADJ_TRASH = N_NODES

INC_K = 56
INC_NNZ_PAD = NW * INC_K * CH
INC_RANGE = N_FACES // NC
INC_NACC = 163840
INC_RPT = INC_NACC // NS
INC_TRASH = N_FACES


def _x1_pass_body(x_ref, w_ref, z_ref, s_ref):
    i = pl.program_id(0)
    x = x_ref[...]
    z_ref[...] = jnp.dot(x, w_ref[...],
                         preferred_element_type=jnp.float32
                         ).astype(jnp.bfloat16)
    @pl.when(i == 0)
    def _():
        s_ref[...] = jnp.zeros_like(s_ref)
    s_ref[...] += jnp.sum(x, axis=0, keepdims=True)


def _x1_pass(x1, w12p):
    rb = 8000
    grid = N_EDGES // rb
    return pl.pallas_call(
        _x1_pass_body,
        grid=(grid,),
        in_specs=[
            pl.BlockSpec((rb, D0), lambda i: (i, 0)),
            pl.BlockSpec((D0, D2P), lambda i: (0, 0)),
        ],
        out_specs=[
            pl.BlockSpec((rb, D2P), lambda i: (i, 0)),
            pl.BlockSpec((1, D0), lambda i: (0, 0)),
        ],
        out_shape=[
            jax.ShapeDtypeStruct((N_EDGES, D2P), jnp.bfloat16),
            jax.ShapeDtypeStruct((1, D0), jnp.float32),
        ],
    )(x1, w12p)


def _mm_body(x_ref, w_ref, o_ref):
    o_ref[...] = jnp.dot(x_ref[...], w_ref[...],
                         preferred_element_type=jnp.float32
                         ).astype(jnp.bfloat16)


def _mm(x, w):
    rb = 2000
    grid = x.shape[0] // rb
    return pl.pallas_call(
        _mm_body,
        grid=(grid,),
        in_specs=[
            pl.BlockSpec((rb, D0), lambda i: (i, 0)),
            pl.BlockSpec((D0, D0), lambda i: (0, 0)),
        ],
        out_specs=pl.BlockSpec((rb, D0), lambda i: (i, 0)),
        out_shape=jax.ShapeDtypeStruct((x.shape[0], D0), jnp.bfloat16),
    )(x, w)


def _mid_body(a_ref, w_ref, o_ref):
    t = jax.nn.relu(a_ref[0].astype(jnp.float32) + a_ref[1].astype(jnp.float32))
    o_ref[...] = jnp.dot(t, w_ref[...], preferred_element_type=jnp.float32
                         ).astype(jnp.bfloat16)


def _mid(a, w):
    rb = 2000
    grid = N_NODES // rb
    return pl.pallas_call(
        _mid_body,
        grid=(grid,),
        in_specs=[
            pl.BlockSpec((2, rb, D0), lambda i: (0, i, 0)),
            pl.BlockSpec((D0, D0), lambda i: (0, 0)),
        ],
        out_specs=pl.BlockSpec((rb, D0), lambda i: (i, 0)),
        out_shape=jax.ShapeDtypeStruct((N_NODES, D0), jnp.bfloat16),
    )(a, w)


def _make_sc_spmm(table_rows, d, k_chunks, stage_k, slots, nacc, rpt, trash,
                  split_range, ch=CH, dtype=jnp.float32, tc_tiling=None):
    mesh = plsc.VectorSubcoreMesh(core_axis_name="c", subcore_axis_name="s")

    def body(table, src2d, dst2d, zeros, out, sidx_v, didx_v, rows_v, accum,
             *sems):
        sem_g = sems[:slots]
        sem_s = sems[slots:]
        c = lax.axis_index("c")
        s = lax.axis_index("s")

        def fire_g(u, chunk):
            pltpu.async_copy(table.at[sidx_v.at[chunk]], rows_v.at[u],
                             sem_g[u])

        def wait_g(u):
            pltpu.make_async_copy(table.at[sidx_v.at[0]], rows_v.at[u],
                                  sem_g[u]).wait()

        def fire_s(u, chunk):
            pltpu.async_copy(rows_v.at[u], accum.at[didx_v.at[chunk]],
                             sem_s[u], add=True)

        def wait_s(u):
            pltpu.make_async_copy(rows_v.at[u], accum.at[didx_v.at[0]],
                                  sem_s[u]).wait()

        pltpu.sync_copy(zeros, accum.at[pl.ds(s * rpt, rpt)])
        plsc.subcore_barrier()

        if split_range:
            row_base = s * k_chunks
        else:
            row_base = (s * NC + c) * k_chunks

        n_steps = stage_k // slots
        for stage in range(k_chunks // stage_k):
            row0 = row_base + stage * stage_k
            pltpu.sync_copy(src2d.at[pl.ds(row0, stage_k)], sidx_v)
            pltpu.sync_copy(dst2d.at[pl.ds(row0, stage_k)], didx_v)

            if split_range:
                lo = c * INC_RANGE
                def remap(j, _):
                    def remap16(q, _):
                        dv = didx_v[j, pl.ds(q * 16, 16)]
                        inr = (dv >= lo) & (dv < lo + INC_RANGE)
                        didx_v[j, pl.ds(q * 16, 16)] = jnp.where(
                            inr, dv - lo, trash + (dv & 1023))
                        return 0
                    return lax.fori_loop(0, ch // 16, remap16, 0)
                lax.fori_loop(0, stage_k, remap, 0)

            for u in range(slots):
                fire_g(u, u)

            def step(t, _):
                base = t * slots
                for u in range(slots):
                    wait_g(u)
                    fire_s(u, base + u)
                for u in range(slots):
                    wait_s(u)
                    @pl.when(t < n_steps - 1)
                    def _():
                        fire_g(u, base + slots + u)
                return 0
            lax.fori_loop(0, n_steps, step, 0)

        plsc.subcore_barrier()

        pltpu.sync_copy(accum.at[pl.ds(s * rpt, rpt)],
                        out.at[pl.ds(c * nacc + s * rpt, rpt)])

    return functools.partial(
        pl.kernel,
        out_type=jax.ShapeDtypeStruct((NC * nacc, d), dtype),
        mesh=mesh,
        compiler_params=pltpu.CompilerParams(
            use_tc_tiling_on_sc=(d == D0 and dtype == jnp.float32
                                 if tc_tiling is None else tc_tiling)),
        scratch_types=[
            pltpu.VMEM((stage_k, ch), jnp.int32),
            pltpu.VMEM((stage_k, ch), jnp.int32),
            pltpu.VMEM((slots, ch, d), dtype),
            pltpu.VMEM_SHARED((nacc, d), dtype),
        ] + [pltpu.SemaphoreType.DMA] * (2 * slots),
    )(body)


_adj_spmm = _make_sc_spmm(N_NODES, D0, ADJ_K, 80, 8, ADJ_NACC, ADJ_RPT,
                          ADJ_TRASH, split_range=False, dtype=jnp.bfloat16)
_inc_spmm = _make_sc_spmm(N_EDGES, D2P, INC_K, INC_K, 8, INC_NACC, INC_RPT,
                          INC_TRASH, split_range=False, dtype=jnp.bfloat16)


def _final_body(a_ref, b_ref, cs1_ref, l0_ref, b0_ref, l1_ref, b1_ref,
                l2_ref, b2_ref, o_ref, s0_ref, s2_ref):
    i = pl.program_id(0)
    n = pl.num_programs(0)

    @pl.when(i == 0)
    def _():
        s0_ref[...] = jnp.zeros_like(s0_ref)
        s2_ref[...] = jnp.zeros_like(s2_ref)

    x0 = jax.nn.relu(a_ref[0].astype(jnp.float32)
                     + a_ref[1].astype(jnp.float32))
    s0_ref[...] += jnp.sum(x0, axis=0, keepdims=True)
    x2 = jax.nn.relu(b_ref[0].astype(jnp.float32)
                     + b_ref[1].astype(jnp.float32))
    s2_ref[...] += jnp.sum(x2, axis=0, keepdims=True)

    @pl.when(i == n - 1)
    def _():
        y0 = jnp.dot(s0_ref[...] * (1.0 / N_NODES), l0_ref[...],
                     preferred_element_type=jnp.float32) + b0_ref[...]
        y1 = jnp.dot(cs1_ref[...] * (1.0 / N_EDGES), l1_ref[...],
                     preferred_element_type=jnp.float32) + b1_ref[...]
        y2 = jnp.dot(s2_ref[...] * (1.0 / N_FACES), l2_ref[...],
                     preferred_element_type=jnp.float32) + b2_ref[...]
        o_ref[...] = y0 + y1 + y2


def _final(a, b, cs1, l0p, b0p, l1p, b1p, l2p, b2p):
    grid = 10
    rb0 = N_NODES // grid
    rb2 = N_FACES // grid
    return pl.pallas_call(
        _final_body,
        grid=(grid,),
        in_specs=[
            pl.BlockSpec((2, rb0, D0), lambda i: (0, i, 0)),
            pl.BlockSpec((2, rb2, D2P), lambda i: (0, i, 0)),
            pl.BlockSpec((1, D0), lambda i: (0, 0)),
            pl.BlockSpec((D0, 128), lambda i: (0, 0)),
            pl.BlockSpec((1, 128), lambda i: (0, 0)),
            pl.BlockSpec((D0, 128), lambda i: (0, 0)),
            pl.BlockSpec((1, 128), lambda i: (0, 0)),
            pl.BlockSpec((D2P, 128), lambda i: (0, 0)),
            pl.BlockSpec((1, 128), lambda i: (0, 0)),
        ],
        out_specs=pl.BlockSpec((1, 128), lambda i: (0, 0)),
        out_shape=jax.ShapeDtypeStruct((1, 128), jnp.float32),
        scratch_shapes=[
            pltpu.VMEM((1, D0), jnp.float32),
            pltpu.VMEM((1, D2P), jnp.float32),
        ],
    )(a, b, cs1, l0p, b0p, l1p, b1p, l2p, b2p)


def _pad_idx(src, dst, nnz_pad, trash_dst, trash_spread, ch=CH):
    n = src.shape[0]
    pad = nnz_pad - n
    src_p = jnp.concatenate([src.astype(jnp.int32),
                             jnp.zeros((pad,), jnp.int32)])
    trash = trash_dst + jnp.arange(pad, dtype=jnp.int32) % trash_spread
    dst_p = jnp.concatenate([dst.astype(jnp.int32), trash])
    return src_p.reshape(-1, ch), dst_p.reshape(-1, ch)


def kernel(x_0, x_1, adjacency_0, incidence_2_t,
           W0_0, W12_0, W0_1, W12_1,
           lin0_w, lin0_b, lin1_w, lin1_b, lin2_w, lin2_b):
    f32 = jnp.float32

    w12p = jnp.zeros((D0, D2P), f32).at[:, :5].set(W12_1)
    z1p, cs1 = _x1_pass(x_1, w12p)

    inc_src, inc_dst = _pad_idx(incidence_2_t[1], incidence_2_t[0],
                                INC_NNZ_PAD, N_FACES, 3840)
    inc_zeros = jnp.zeros((INC_RPT, D2P), jnp.bfloat16)
    x2acc = _inc_spmm(z1p, inc_src, inc_dst, inc_zeros)

    adj_src, adj_dst = _pad_idx(adjacency_0[1], adjacency_0[0],
                                ADJ_NNZ_PAD, ADJ_TRASH, 192)
    adj_zeros = jnp.zeros((ADJ_RPT, D0), jnp.bfloat16)

    z0 = _mm(x_0, W0_0)
    a1 = _adj_spmm(z0, adj_src, adj_dst, adj_zeros)
    h = _mid(a1.reshape(2, ADJ_NACC, D0), W0_1)
    a2 = _adj_spmm(h, adj_src, adj_dst, adj_zeros)

    def padw(w, rows):
        wp = jnp.zeros((rows, 128), f32)
        return wp.at[:w.shape[0], :2].set(w)

    def padb(b):
        return jnp.zeros((1, 128), f32).at[0, :2].set(b)

    out = _final(a2.reshape(2, ADJ_NACC, D0),
                 x2acc.reshape(2, INC_NACC, D2P),
                 cs1,
                 padw(lin0_w, D0), padb(lin0_b),
                 padw(lin1_w, D0), padb(lin1_b),
                 padw(lin2_w, D2P), padb(lin2_b))
    return out[0, :2]
